# Initial kernel scaffold; baseline (speedup 1.0000x reference)
import jax
import jax.numpy as jnp
from jax.experimental import pallas as pl


def kernel(x, edge_index, edge_attr, node2orig, W_in, b_in, W_edge, b_edge, W_mlp, b_mlp, W_out, b_out):
    raise NotImplementedError("write your pallas kernel here")


# speedup vs baseline: 1.0299x; 1.0299x over previous
import functools

import jax
import jax.numpy as jnp
from jax import lax
from jax.experimental import pallas as pl
from jax.experimental.pallas import tpu as pltpu
from jax.experimental.pallas import tpu_sc as plsc

F32 = jnp.float32

N = 50000
NORIG = 10000
E = 320000
D = 128

NC = 2
NS = 16
NW = NC * NS

EB = 128
E_PAD = 327680
NBLK_E = E_PAD // EB
GBPW = NBLK_E // NW
SBPS = NBLK_E // NS
CW = 32
NPS = N // NS

N_PAD = 65536
NBLK_P = N_PAD // EB
PBPS = NBLK_P // NW
NPOOL = 10240
POOL_PS = NPOOL // NS

_SC_PARAMS = pltpu.CompilerParams(use_tc_tiling_on_sc=False)

@functools.cache
def _sc_mesh():
    return plsc.VectorSubcoreMesh(core_axis_name="c", subcore_axis_name="s",
                                  num_cores=NC, num_subcores=NS)


def _gather_body(table_hbm, idx_hbm, out_hbm, idxb, rows, sem):
    c = lax.axis_index("c")
    s = lax.axis_index("s")
    wid = s * NC + c
    base = wid * GBPW
    pltpu.sync_copy(idx_hbm.at[pl.ds(base, GBPW)], idxb)

    @pl.loop(0, GBPW)
    def _(i):
        pltpu.async_copy(table_hbm.at[idxb.at[i]], rows, sem).wait()
        pltpu.sync_copy(rows, out_hbm.at[pl.ds((base + i) * EB, EB)])


def _sc_gather(table, idx2d):
    k = pl.kernel(
        _gather_body,
        out_type=jax.ShapeDtypeStruct((E_PAD, D), F32),
        mesh=_sc_mesh(),
        compiler_params=_SC_PARAMS,
        scratch_types=[
            pltpu.VMEM((GBPW, EB), jnp.int32),
            pltpu.VMEM((EB, D), F32),
            pltpu.SemaphoreType.DMA,
        ],
    )
    return k(table, idx2d)


def _scatter_body(m_hbm, dst_hbm, h_hbm, z_hbm, idxb, mbuf, acc):
    c = lax.axis_index("c")
    s = lax.axis_index("s")
    base = s * SBPS
    pltpu.sync_copy(dst_hbm.at[pl.ds(base, SBPS)], idxb)
    for j in range(2):
        ch = 2 * c + j
        col = ch * CW
        pltpu.sync_copy(
            h_hbm.at[pl.ds(s * NPS, NPS), pl.ds(col, CW)],
            acc.at[pl.ds(s * NPS, NPS)],
        )
        plsc.subcore_barrier()

        @pl.loop(0, SBPS)
        def _(i):
            pltpu.sync_copy(
                m_hbm.at[pl.ds((base + i) * EB, EB), pl.ds(col, CW)], mbuf
            )
            pltpu.sync_copy(mbuf, acc.at[idxb.at[i]], add=True)

        plsc.subcore_barrier()
        pltpu.sync_copy(
            acc.at[pl.ds(s * NPS, NPS)],
            z_hbm.at[pl.ds(s * NPS, NPS), pl.ds(col, CW)],
        )
        plsc.subcore_barrier()


def _sc_scatter_z(m, dst2d, h, out_rows):
    k = pl.kernel(
        _scatter_body,
        out_type=jax.ShapeDtypeStruct((out_rows, D), F32),
        mesh=_sc_mesh(),
        compiler_params=_SC_PARAMS,
        scratch_types=[
            pltpu.VMEM((SBPS, EB), jnp.int32),
            pltpu.VMEM((EB, CW), F32),
            pltpu.VMEM_SHARED((N, CW), F32),
        ],
    )
    return k(m, dst2d, h)


def _pool_body(hp_hbm, idx_hbm, sums_hbm, cnts_hbm,
               idxb, hbuf, zbuf, zcbuf, obuf, accS, accC):
    c = lax.axis_index("c")
    s = lax.axis_index("s")

    @pl.loop(0, EB)
    def _(i):
        @pl.loop(0, D // 16)
        def _(j):
            zbuf[pl.ds(i, 1), pl.ds(j * 16, 16)] = jnp.zeros((1, 16), F32)

    @pl.loop(0, EB)
    def _(i):
        zcbuf[pl.ds(i, 1), pl.ds(0, 16)] = jnp.zeros((1, 16), F32)
        obuf[pl.ds(i, 1), pl.ds(0, 16)] = jnp.ones((1, 16), F32)

    @pl.loop(0, POOL_PS // EB)
    def _(i):
        pltpu.sync_copy(zbuf, accS.at[pl.ds(s * POOL_PS + i * EB, EB)])
        pltpu.sync_copy(zcbuf, accC.at[pl.ds(s * POOL_PS + i * EB, EB)])
    plsc.subcore_barrier()

    base = (c * NS + s) * PBPS
    pltpu.sync_copy(idx_hbm.at[pl.ds(base, PBPS)], idxb)

    @pl.loop(0, PBPS)
    def _(i):
        pltpu.sync_copy(hp_hbm.at[pl.ds((base + i) * EB, EB)], hbuf)
        pltpu.sync_copy(hbuf, accS.at[idxb.at[i]], add=True)
        pltpu.sync_copy(obuf, accC.at[idxb.at[i]], add=True)

    plsc.subcore_barrier()
    pltpu.sync_copy(
        accS.at[pl.ds(s * POOL_PS, POOL_PS)],
        sums_hbm.at[c].at[pl.ds(s * POOL_PS, POOL_PS)],
    )
    pltpu.sync_copy(
        accC.at[pl.ds(s * POOL_PS, POOL_PS)],
        cnts_hbm.at[c].at[pl.ds(s * POOL_PS, POOL_PS)],
    )


def _sc_pool(hp, idx2d):
    k = pl.kernel(
        _pool_body,
        out_type=(
            jax.ShapeDtypeStruct((NC, NPOOL, D), F32),
            jax.ShapeDtypeStruct((NC, NPOOL, 16), F32),
        ),
        mesh=_sc_mesh(),
        compiler_params=_SC_PARAMS,
        scratch_types=[
            pltpu.VMEM((PBPS, EB), jnp.int32),
            pltpu.VMEM((EB, D), F32),
            pltpu.VMEM((EB, D), F32),
            pltpu.VMEM((EB, 16), F32),
            pltpu.VMEM((EB, 16), F32),
            pltpu.VMEM_SHARED((NPOOL, D), F32),
            pltpu.VMEM_SHARED((NPOOL, 16), F32),
        ],
    )
    return k(hp, idx2d)


def _mm_bias_body(x_ref, w_ref, b_ref, o_ref):
    o_ref[...] = (
        jnp.dot(x_ref[...], w_ref[...], preferred_element_type=F32) + b_ref[...]
    )


def _tc_mm_bias(x, w, b, blk):
    rows = x.shape[0]
    return pl.pallas_call(
        _mm_bias_body,
        grid=(rows // blk,),
        in_specs=[
            pl.BlockSpec((blk, x.shape[1]), lambda i: (i, 0)),
            pl.BlockSpec(w.shape, lambda i: (0, 0)),
            pl.BlockSpec((1, w.shape[1]), lambda i: (0, 0)),
        ],
        out_specs=pl.BlockSpec((blk, w.shape[1]), lambda i: (i, 0)),
        out_shape=jax.ShapeDtypeStruct((rows, w.shape[1]), F32),
    )(x, w, b)


def _edge_proj_body(a_ref, w_ref, b_ref, o_ref):
    o_ref[...] = jnp.maximum(
        jnp.dot(a_ref[...], w_ref[...], preferred_element_type=F32) + b_ref[...],
        0.0,
    )


def _tc_edge_proj(attr_pad, w_all, b_all):
    blk = 512
    return pl.pallas_call(
        _edge_proj_body,
        grid=(E_PAD // blk,),
        in_specs=[
            pl.BlockSpec((blk, attr_pad.shape[1]), lambda i: (i, 0)),
            pl.BlockSpec(w_all.shape, lambda i: (0, 0)),
            pl.BlockSpec((1, w_all.shape[1]), lambda i: (0, 0)),
        ],
        out_specs=pl.BlockSpec((blk, w_all.shape[1]), lambda i: (i, 0)),
        out_shape=jax.ShapeDtypeStruct((E_PAD, w_all.shape[1]), F32),
    )(attr_pad, w_all, b_all)


def _msg_body(nreal_blocks, g_ref, e_ref, o_ref):
    v = jnp.maximum(g_ref[...] + e_ref[...], 0.0)
    o_ref[...] = jnp.where(pl.program_id(0) < nreal_blocks, v, 0.0)


def _tc_messages(g, e_all, layer):
    blk = 512
    nreal = E // blk
    return pl.pallas_call(
        functools.partial(_msg_body, nreal),
        grid=(E_PAD // blk,),
        in_specs=[
            pl.BlockSpec((blk, D), lambda i: (i, 0)),
            pl.BlockSpec((blk, D), lambda i, L=layer: (i, L)),
        ],
        out_specs=pl.BlockSpec((blk, D), lambda i: (i, 0)),
        out_shape=jax.ShapeDtypeStruct((E_PAD, D), F32),
    )(g, e_all)


def _mlp_body(z_ref, w1_ref, b1_ref, w2_ref, b2_ref, o_ref):
    t = jnp.maximum(
        jnp.dot(z_ref[...], w1_ref[...], preferred_element_type=F32)
        + b1_ref[...],
        0.0,
    )
    o_ref[...] = jnp.maximum(
        jnp.dot(t, w2_ref[...], preferred_element_type=F32) + b2_ref[...],
        0.0,
    )


def _tc_mlp(z, w1, b1, w2, b2, blk):
    rows = z.shape[0]
    return pl.pallas_call(
        _mlp_body,
        grid=(rows // blk,),
        in_specs=[
            pl.BlockSpec((blk, D), lambda i: (i, 0)),
            pl.BlockSpec((D, D), lambda i: (0, 0)),
            pl.BlockSpec((1, D), lambda i: (0, 0)),
            pl.BlockSpec((D, D), lambda i: (0, 0)),
            pl.BlockSpec((1, D), lambda i: (0, 0)),
        ],
        out_specs=pl.BlockSpec((blk, D), lambda i: (i, 0)),
        out_shape=jax.ShapeDtypeStruct((rows, D), F32),
    )(z, w1, b1, w2, b2)


def _final_body(s_ref, c_ref, w_ref, b_ref, o_ref):
    ssum = s_ref[0] + s_ref[1]
    cnt = c_ref[0, :, 0:1] + c_ref[1, :, 0:1]
    pooled = ssum / jnp.maximum(cnt, 1.0)
    o_ref[...] = (
        jnp.dot(pooled, w_ref[...], preferred_element_type=F32) + b_ref[...]
    )


def _tc_final(sums, cnts, w_out, b_out):
    blk = 400
    return pl.pallas_call(
        _final_body,
        grid=(NORIG // blk,),
        in_specs=[
            pl.BlockSpec((NC, blk, D), lambda i: (0, i, 0)),
            pl.BlockSpec((NC, blk, 16), lambda i: (0, i, 0)),
            pl.BlockSpec((D, D), lambda i: (0, 0)),
            pl.BlockSpec((1, D), lambda i: (0, 0)),
        ],
        out_specs=pl.BlockSpec((blk, D), lambda i: (i, 0)),
        out_shape=jax.ShapeDtypeStruct((NORIG, D), F32),
    )(sums, cnts, w_out, b_out)


def kernel(x, edge_index, edge_attr, node2orig, W_in, b_in, W_edge, b_edge,
           W_mlp, b_mlp, W_out, b_out):
    n_layers = W_edge.shape[0]

    pad_e = E_PAD - E
    src = jnp.concatenate(
        [edge_index[0], jnp.zeros((pad_e,), jnp.int32)]).reshape(NBLK_E, EB)
    dst = jnp.concatenate(
        [edge_index[1], jnp.zeros((pad_e,), jnp.int32)]).reshape(NBLK_E, EB)
    attr_pad = jnp.concatenate(
        [edge_attr, jnp.zeros((pad_e, edge_attr.shape[1]), F32)])

    n2o = jnp.concatenate(
        [node2orig, jnp.full((N_PAD - N,), NORIG, jnp.int32)]).reshape(
            NBLK_P, EB)

    w_edge_all = W_edge.transpose(1, 0, 2).reshape(W_edge.shape[1],
                                                   n_layers * D)
    b_edge_all = b_edge.reshape(1, n_layers * D)

    h = _tc_mm_bias(x, W_in, b_in.reshape(1, D), 400)
    e_all = _tc_edge_proj(attr_pad, w_edge_all, b_edge_all)

    for l in range(n_layers):
        g = _sc_gather(h, src)
        m = _tc_messages(g, e_all, l)
        out_rows = N if l < n_layers - 1 else N_PAD
        z = _sc_scatter_z(m, dst, h, out_rows)
        blk = 400 if l < n_layers - 1 else 512
        h = _tc_mlp(z, W_mlp[l, 0], b_mlp[l, 0].reshape(1, D),
                    W_mlp[l, 1], b_mlp[l, 1].reshape(1, D), blk)

    sums, cnts = _sc_pool(h, n2o)
    return _tc_final(sums, cnts, W_out, b_out.reshape(1, D))


# speedup vs baseline: 1.1678x; 1.1339x over previous
import functools

import jax
import jax.numpy as jnp
from jax import lax
from jax.experimental import pallas as pl
from jax.experimental.pallas import tpu as pltpu
from jax.experimental.pallas import tpu_sc as plsc

F32 = jnp.float32

N = 50000
NORIG = 10000
E = 320000
D = 128

NC = 2
NS = 16
NW = NC * NS

EB = 128
E_PAD = 327680
NBLK_E = E_PAD // EB
GBPW = NBLK_E // NW
SBPS = NBLK_E // NS
CW = 32
NPS = N // NS

N_PAD = 65536
NBLK_P = N_PAD // EB
PBPS = NBLK_P // NW
NPOOL = 10240
POOL_PS = NPOOL // NS

_SC_PARAMS = pltpu.CompilerParams(use_tc_tiling_on_sc=False)

@functools.cache
def _sc_mesh():
    return plsc.VectorSubcoreMesh(core_axis_name="c", subcore_axis_name="s",
                                  num_cores=NC, num_subcores=NS)


def _gather_body(table_hbm, idx_hbm, out_hbm, idxb, rowsA, rowsB, semA, semB):
    c = lax.axis_index("c")
    s = lax.axis_index("s")
    wid = s * NC + c
    base = wid * GBPW
    pltpu.sync_copy(idx_hbm.at[pl.ds(base, GBPW)], idxb)

    pltpu.async_copy(table_hbm.at[idxb.at[0]], rowsA, semA)
    pltpu.async_copy(table_hbm.at[idxb.at[1]], rowsB, semB)

    @pl.loop(0, GBPW - 2, step=2)
    def _(i):
        pltpu.make_async_copy(table_hbm.at[idxb.at[i]], rowsA, semA).wait()
        pltpu.sync_copy(rowsA, out_hbm.at[pl.ds((base + i) * EB, EB)])
        pltpu.async_copy(table_hbm.at[idxb.at[i + 2]], rowsA, semA)
        pltpu.make_async_copy(table_hbm.at[idxb.at[i + 1]], rowsB, semB).wait()
        pltpu.sync_copy(rowsB, out_hbm.at[pl.ds((base + i + 1) * EB, EB)])
        pltpu.async_copy(table_hbm.at[idxb.at[i + 3]], rowsB, semB)

    i = GBPW - 2
    pltpu.make_async_copy(table_hbm.at[idxb.at[i]], rowsA, semA).wait()
    pltpu.sync_copy(rowsA, out_hbm.at[pl.ds((base + i) * EB, EB)])
    pltpu.make_async_copy(table_hbm.at[idxb.at[i + 1]], rowsB, semB).wait()
    pltpu.sync_copy(rowsB, out_hbm.at[pl.ds((base + i + 1) * EB, EB)])


def _sc_gather(table, idx2d):
    k = pl.kernel(
        _gather_body,
        out_type=jax.ShapeDtypeStruct((E_PAD, D), F32),
        mesh=_sc_mesh(),
        compiler_params=_SC_PARAMS,
        scratch_types=[
            pltpu.VMEM((GBPW, EB), jnp.int32),
            pltpu.VMEM((EB, D), F32),
            pltpu.VMEM((EB, D), F32),
            pltpu.SemaphoreType.DMA,
            pltpu.SemaphoreType.DMA,
        ],
    )
    return k(table, idx2d)


def _scatter_body(m_hbm, dst_hbm, h_hbm, z_hbm, idxb, mbufA, mbufB,
                  semA, semB, acc):
    c = lax.axis_index("c")
    s = lax.axis_index("s")
    base = s * SBPS
    pltpu.sync_copy(dst_hbm.at[pl.ds(base, SBPS)], idxb)
    for j in range(2):
        ch = 2 * c + j
        col = ch * CW
        pltpu.sync_copy(
            h_hbm.at[pl.ds(s * NPS, NPS), pl.ds(col, CW)],
            acc.at[pl.ds(s * NPS, NPS)],
        )
        plsc.subcore_barrier()

        def _mref(i):
            return m_hbm.at[pl.ds((base + i) * EB, EB), pl.ds(col, CW)]

        pltpu.async_copy(_mref(0), mbufA, semA)
        pltpu.async_copy(_mref(1), mbufB, semB)

        @pl.loop(0, SBPS - 2, step=2)
        def _(i):
            pltpu.make_async_copy(_mref(i), mbufA, semA).wait()
            pltpu.sync_copy(mbufA, acc.at[idxb.at[i]], add=True)
            pltpu.async_copy(_mref(i + 2), mbufA, semA)
            pltpu.make_async_copy(_mref(i + 1), mbufB, semB).wait()
            pltpu.sync_copy(mbufB, acc.at[idxb.at[i + 1]], add=True)
            pltpu.async_copy(_mref(i + 3), mbufB, semB)

        i = SBPS - 2
        pltpu.make_async_copy(_mref(i), mbufA, semA).wait()
        pltpu.sync_copy(mbufA, acc.at[idxb.at[i]], add=True)
        pltpu.make_async_copy(_mref(i + 1), mbufB, semB).wait()
        pltpu.sync_copy(mbufB, acc.at[idxb.at[i + 1]], add=True)

        plsc.subcore_barrier()
        pltpu.sync_copy(
            acc.at[pl.ds(s * NPS, NPS)],
            z_hbm.at[pl.ds(s * NPS, NPS), pl.ds(col, CW)],
        )
        plsc.subcore_barrier()


def _sc_scatter_z(m, dst2d, h, out_rows):
    k = pl.kernel(
        _scatter_body,
        out_type=jax.ShapeDtypeStruct((out_rows, D), F32),
        mesh=_sc_mesh(),
        compiler_params=_SC_PARAMS,
        scratch_types=[
            pltpu.VMEM((SBPS, EB), jnp.int32),
            pltpu.VMEM((EB, CW), F32),
            pltpu.VMEM((EB, CW), F32),
            pltpu.SemaphoreType.DMA,
            pltpu.SemaphoreType.DMA,
            pltpu.VMEM_SHARED((N, CW), F32),
        ],
    )
    return k(m, dst2d, h)


def _pool_body(hp_hbm, idx_hbm, sums_hbm, cnts_hbm,
               idxb, hbuf, zbuf, zcbuf, obuf, semA, semB, accS, accC):
    c = lax.axis_index("c")
    s = lax.axis_index("s")

    @pl.loop(0, EB)
    def _(i):
        @pl.loop(0, D // 16)
        def _(j):
            zbuf[pl.ds(i, 1), pl.ds(j * 16, 16)] = jnp.zeros((1, 16), F32)

    @pl.loop(0, EB)
    def _(i):
        zcbuf[pl.ds(i, 1), pl.ds(0, 16)] = jnp.zeros((1, 16), F32)
        obuf[pl.ds(i, 1), pl.ds(0, 16)] = jnp.ones((1, 16), F32)

    @pl.loop(0, POOL_PS // EB)
    def _(i):
        pltpu.sync_copy(zbuf, accS.at[pl.ds(s * POOL_PS + i * EB, EB)])
        pltpu.sync_copy(zcbuf, accC.at[pl.ds(s * POOL_PS + i * EB, EB)])
    plsc.subcore_barrier()
    hbuf2 = zbuf

    base = (c * NS + s) * PBPS
    pltpu.sync_copy(idx_hbm.at[pl.ds(base, PBPS)], idxb)

    def _href(i):
        return hp_hbm.at[pl.ds((base + i) * EB, EB)]

    pltpu.async_copy(_href(0), hbuf, semA)
    pltpu.async_copy(_href(1), hbuf2, semB)

    @pl.loop(0, PBPS - 2, step=2)
    def _(i):
        pltpu.make_async_copy(_href(i), hbuf, semA).wait()
        pltpu.sync_copy(hbuf, accS.at[idxb.at[i]], add=True)
        pltpu.sync_copy(obuf, accC.at[idxb.at[i]], add=True)
        pltpu.async_copy(_href(i + 2), hbuf, semA)
        pltpu.make_async_copy(_href(i + 1), hbuf2, semB).wait()
        pltpu.sync_copy(hbuf2, accS.at[idxb.at[i + 1]], add=True)
        pltpu.sync_copy(obuf, accC.at[idxb.at[i + 1]], add=True)
        pltpu.async_copy(_href(i + 3), hbuf2, semB)

    i = PBPS - 2
    pltpu.make_async_copy(_href(i), hbuf, semA).wait()
    pltpu.sync_copy(hbuf, accS.at[idxb.at[i]], add=True)
    pltpu.sync_copy(obuf, accC.at[idxb.at[i]], add=True)
    pltpu.make_async_copy(_href(i + 1), hbuf2, semB).wait()
    pltpu.sync_copy(hbuf2, accS.at[idxb.at[i + 1]], add=True)
    pltpu.sync_copy(obuf, accC.at[idxb.at[i + 1]], add=True)

    plsc.subcore_barrier()
    pltpu.sync_copy(
        accS.at[pl.ds(s * POOL_PS, POOL_PS)],
        sums_hbm.at[c].at[pl.ds(s * POOL_PS, POOL_PS)],
    )
    pltpu.sync_copy(
        accC.at[pl.ds(s * POOL_PS, POOL_PS)],
        cnts_hbm.at[c].at[pl.ds(s * POOL_PS, POOL_PS)],
    )


def _sc_pool(hp, idx2d):
    k = pl.kernel(
        _pool_body,
        out_type=(
            jax.ShapeDtypeStruct((NC, NPOOL, D), F32),
            jax.ShapeDtypeStruct((NC, NPOOL, 16), F32),
        ),
        mesh=_sc_mesh(),
        compiler_params=_SC_PARAMS,
        scratch_types=[
            pltpu.VMEM((PBPS, EB), jnp.int32),
            pltpu.VMEM((EB, D), F32),
            pltpu.VMEM((EB, D), F32),
            pltpu.VMEM((EB, 16), F32),
            pltpu.VMEM((EB, 16), F32),
            pltpu.SemaphoreType.DMA,
            pltpu.SemaphoreType.DMA,
            pltpu.VMEM_SHARED((NPOOL, D), F32),
            pltpu.VMEM_SHARED((NPOOL, 16), F32),
        ],
    )
    return k(hp, idx2d)


def _mm_bias_body(x_ref, w_ref, b_ref, o_ref):
    o_ref[...] = (
        jnp.dot(x_ref[...], w_ref[...], preferred_element_type=F32) + b_ref[...]
    )


def _tc_mm_bias(x, w, b, blk):
    rows = x.shape[0]
    return pl.pallas_call(
        _mm_bias_body,
        grid=(rows // blk,),
        in_specs=[
            pl.BlockSpec((blk, x.shape[1]), lambda i: (i, 0)),
            pl.BlockSpec(w.shape, lambda i: (0, 0)),
            pl.BlockSpec((1, w.shape[1]), lambda i: (0, 0)),
        ],
        out_specs=pl.BlockSpec((blk, w.shape[1]), lambda i: (i, 0)),
        out_shape=jax.ShapeDtypeStruct((rows, w.shape[1]), F32),
    )(x, w, b)


def _edge_proj_body(a_ref, w_ref, b_ref, o_ref):
    o_ref[...] = jnp.maximum(
        jnp.dot(a_ref[...], w_ref[...], preferred_element_type=F32) + b_ref[...],
        0.0,
    )


def _tc_edge_proj(attr_pad, w_all, b_all):
    blk = 512
    return pl.pallas_call(
        _edge_proj_body,
        grid=(E_PAD // blk,),
        in_specs=[
            pl.BlockSpec((blk, attr_pad.shape[1]), lambda i: (i, 0)),
            pl.BlockSpec(w_all.shape, lambda i: (0, 0)),
            pl.BlockSpec((1, w_all.shape[1]), lambda i: (0, 0)),
        ],
        out_specs=pl.BlockSpec((blk, w_all.shape[1]), lambda i: (i, 0)),
        out_shape=jax.ShapeDtypeStruct((E_PAD, w_all.shape[1]), F32),
    )(attr_pad, w_all, b_all)


def _msg_body(nreal_blocks, g_ref, e_ref, o_ref):
    v = jnp.maximum(g_ref[...] + e_ref[...], 0.0)
    o_ref[...] = jnp.where(pl.program_id(0) < nreal_blocks, v, 0.0)


def _tc_messages(g, e_all, layer):
    blk = 512
    nreal = E // blk
    return pl.pallas_call(
        functools.partial(_msg_body, nreal),
        grid=(E_PAD // blk,),
        in_specs=[
            pl.BlockSpec((blk, D), lambda i: (i, 0)),
            pl.BlockSpec((blk, D), lambda i, L=layer: (i, L)),
        ],
        out_specs=pl.BlockSpec((blk, D), lambda i: (i, 0)),
        out_shape=jax.ShapeDtypeStruct((E_PAD, D), F32),
    )(g, e_all)


def _mlp_body(z_ref, w1_ref, b1_ref, w2_ref, b2_ref, o_ref):
    t = jnp.maximum(
        jnp.dot(z_ref[...], w1_ref[...], preferred_element_type=F32)
        + b1_ref[...],
        0.0,
    )
    o_ref[...] = jnp.maximum(
        jnp.dot(t, w2_ref[...], preferred_element_type=F32) + b2_ref[...],
        0.0,
    )


def _tc_mlp(z, w1, b1, w2, b2, blk):
    rows = z.shape[0]
    return pl.pallas_call(
        _mlp_body,
        grid=(rows // blk,),
        in_specs=[
            pl.BlockSpec((blk, D), lambda i: (i, 0)),
            pl.BlockSpec((D, D), lambda i: (0, 0)),
            pl.BlockSpec((1, D), lambda i: (0, 0)),
            pl.BlockSpec((D, D), lambda i: (0, 0)),
            pl.BlockSpec((1, D), lambda i: (0, 0)),
        ],
        out_specs=pl.BlockSpec((blk, D), lambda i: (i, 0)),
        out_shape=jax.ShapeDtypeStruct((rows, D), F32),
    )(z, w1, b1, w2, b2)


def _final_body(s_ref, c_ref, w_ref, b_ref, o_ref):
    ssum = s_ref[0] + s_ref[1]
    cnt = c_ref[0, :, 0:1] + c_ref[1, :, 0:1]
    pooled = ssum / jnp.maximum(cnt, 1.0)
    o_ref[...] = (
        jnp.dot(pooled, w_ref[...], preferred_element_type=F32) + b_ref[...]
    )


def _tc_final(sums, cnts, w_out, b_out):
    blk = 400
    return pl.pallas_call(
        _final_body,
        grid=(NORIG // blk,),
        in_specs=[
            pl.BlockSpec((NC, blk, D), lambda i: (0, i, 0)),
            pl.BlockSpec((NC, blk, 16), lambda i: (0, i, 0)),
            pl.BlockSpec((D, D), lambda i: (0, 0)),
            pl.BlockSpec((1, D), lambda i: (0, 0)),
        ],
        out_specs=pl.BlockSpec((blk, D), lambda i: (i, 0)),
        out_shape=jax.ShapeDtypeStruct((NORIG, D), F32),
    )(sums, cnts, w_out, b_out)


def kernel(x, edge_index, edge_attr, node2orig, W_in, b_in, W_edge, b_edge,
           W_mlp, b_mlp, W_out, b_out):
    n_layers = W_edge.shape[0]

    pad_e = E_PAD - E
    src = jnp.concatenate(
        [edge_index[0], jnp.zeros((pad_e,), jnp.int32)]).reshape(NBLK_E, EB)
    dst = jnp.concatenate(
        [edge_index[1], jnp.zeros((pad_e,), jnp.int32)]).reshape(NBLK_E, EB)
    attr_pad = jnp.concatenate(
        [edge_attr, jnp.zeros((pad_e, edge_attr.shape[1]), F32)])

    n2o = jnp.concatenate(
        [node2orig, jnp.full((N_PAD - N,), NORIG, jnp.int32)]).reshape(
            NBLK_P, EB)

    w_edge_all = W_edge.transpose(1, 0, 2).reshape(W_edge.shape[1],
                                                   n_layers * D)
    b_edge_all = b_edge.reshape(1, n_layers * D)

    h = _tc_mm_bias(x, W_in, b_in.reshape(1, D), 400)
    e_all = _tc_edge_proj(attr_pad, w_edge_all, b_edge_all)

    for l in range(n_layers):
        g = _sc_gather(h, src)
        m = _tc_messages(g, e_all, l)
        out_rows = N if l < n_layers - 1 else N_PAD
        z = _sc_scatter_z(m, dst, h, out_rows)
        blk = 400 if l < n_layers - 1 else 512
        h = _tc_mlp(z, W_mlp[l, 0], b_mlp[l, 0].reshape(1, D),
                    W_mlp[l, 1], b_mlp[l, 1].reshape(1, D), blk)

    sums, cnts = _sc_pool(h, n2o)
    return _tc_final(sums, cnts, W_out, b_out.reshape(1, D))


# speedup vs baseline: 1.1740x; 1.0053x over previous
import functools

import jax
import jax.numpy as jnp
from jax import lax
from jax.experimental import pallas as pl
from jax.experimental.pallas import tpu as pltpu
from jax.experimental.pallas import tpu_sc as plsc

F32 = jnp.float32

N = 50000
NORIG = 10000
E = 320000
D = 128

NC = 2
NS = 16
NW = NC * NS

EB = 128
E_PAD = 327680
NBLK_E = E_PAD // EB
GBPW = NBLK_E // NW
SBPS = NBLK_E // NS
CW = 32
NPS = N // NS

N_PAD = 65536
NBLK_P = N_PAD // EB
PBPS = NBLK_P // NW
NPOOL = 10240
POOL_PS = NPOOL // NS

_SC_PARAMS = pltpu.CompilerParams(use_tc_tiling_on_sc=False)

@functools.cache
def _sc_mesh():
    return plsc.VectorSubcoreMesh(core_axis_name="c", subcore_axis_name="s",
                                  num_cores=NC, num_subcores=NS)


GNB = 4
GROUNDS = GBPW // GNB


def _gather_body(table_hbm, idx_hbm, out_hbm, idxb, b0, b1, b2, b3,
                 rs0, rs1, ws0, ws1):
    c = lax.axis_index("c")
    s = lax.axis_index("s")
    wid = s * NC + c
    base = wid * GBPW
    pltpu.sync_copy(idx_hbm.at[pl.ds(base, GBPW)], idxb)

    bufs = (b0, b1, b2, b3)
    rsems = (rs0, rs0, rs1, rs1)
    wsems = (ws0, ws0, ws1, ws1)

    def _gref(i):
        return table_hbm.at[idxb.at[i]]

    def _oref(i):
        return out_hbm.at[pl.ds((base + i) * EB, EB)]

    for b in range(GNB):
        pltpu.async_copy(_gref(b), bufs[b], rsems[b])

    @pl.loop(0, GROUNDS - 1)
    def _(r):
        k = r * GNB
        for b in range(GNB):
            pltpu.make_async_copy(_gref(k + b), bufs[b], rsems[b]).wait()
            pltpu.async_copy(bufs[b], _oref(k + b), wsems[b])
        for b in range(GNB):
            pltpu.make_async_copy(bufs[b], _oref(k + b), wsems[b]).wait()
            pltpu.async_copy(_gref(k + GNB + b), bufs[b], rsems[b])

    k = (GROUNDS - 1) * GNB
    for b in range(GNB):
        pltpu.make_async_copy(_gref(k + b), bufs[b], rsems[b]).wait()
        pltpu.async_copy(bufs[b], _oref(k + b), wsems[b])
    for b in range(GNB):
        pltpu.make_async_copy(bufs[b], _oref(k + b), wsems[b]).wait()


def _sc_gather(table, idx2d):
    k = pl.kernel(
        _gather_body,
        out_type=jax.ShapeDtypeStruct((E_PAD, D), F32),
        mesh=_sc_mesh(),
        compiler_params=_SC_PARAMS,
        scratch_types=[
            pltpu.VMEM((GBPW, EB), jnp.int32),
            pltpu.VMEM((EB, D), F32),
            pltpu.VMEM((EB, D), F32),
            pltpu.VMEM((EB, D), F32),
            pltpu.VMEM((EB, D), F32),
            pltpu.SemaphoreType.DMA,
            pltpu.SemaphoreType.DMA,
            pltpu.SemaphoreType.DMA,
            pltpu.SemaphoreType.DMA,
        ],
    )
    return k(table, idx2d)


SEB = 64
SBLK = E_PAD // SEB
SBPS2 = SBLK // NS
SNB = 4
SROUNDS = SBPS2 // SNB


def _scatter_body(m_hbm, dst_hbm, h_hbm, z_hbm, idxb,
                  b0, b1, b2, b3,
                  rs0, rs1, as0, as1, acc):
    c = lax.axis_index("c")
    s = lax.axis_index("s")
    base = s * SBPS2
    pltpu.sync_copy(dst_hbm.at[pl.ds(base, SBPS2)], idxb)
    bufs = (b0, b1, b2, b3)
    rsems = (rs0, rs0, rs1, rs1)
    asems = (as0, as0, as1, as1)
    for j in range(2):
        ch = 2 * c + j
        col = ch * CW
        pltpu.sync_copy(
            h_hbm.at[pl.ds(s * NPS, NPS), pl.ds(col, CW)],
            acc.at[pl.ds(s * NPS, NPS)],
        )
        plsc.subcore_barrier()

        def _mref(i):
            return m_hbm.at[pl.ds((base + i) * SEB, SEB), pl.ds(col, CW)]

        def _aref(i):
            return acc.at[idxb.at[i]]

        for b in range(SNB):
            pltpu.async_copy(_mref(b), bufs[b], rsems[b])

        @pl.loop(0, SROUNDS - 1)
        def _(r):
            k = r * SNB
            for b in range(SNB):
                pltpu.make_async_copy(_mref(k + b), bufs[b], rsems[b]).wait()
                pltpu.async_copy(bufs[b], _aref(k + b), asems[b], add=True)
            for b in range(SNB):
                pltpu.make_async_copy(bufs[b], _aref(k + b), asems[b]).wait()
                pltpu.async_copy(_mref(k + SNB + b), bufs[b], rsems[b])

        k = (SROUNDS - 1) * SNB
        for b in range(SNB):
            pltpu.make_async_copy(_mref(k + b), bufs[b], rsems[b]).wait()
            pltpu.async_copy(bufs[b], _aref(k + b), asems[b], add=True)
        for b in range(SNB):
            pltpu.make_async_copy(bufs[b], _aref(k + b), asems[b]).wait()

        plsc.subcore_barrier()
        pltpu.sync_copy(
            acc.at[pl.ds(s * NPS, NPS)],
            z_hbm.at[pl.ds(s * NPS, NPS), pl.ds(col, CW)],
        )
        plsc.subcore_barrier()


def _sc_scatter_z(m, dst2d, h, out_rows):
    k = pl.kernel(
        _scatter_body,
        out_type=jax.ShapeDtypeStruct((out_rows, D), F32),
        mesh=_sc_mesh(),
        compiler_params=_SC_PARAMS,
        scratch_types=[
            pltpu.VMEM((SBPS2, SEB), jnp.int32),
        ] + [pltpu.VMEM((SEB, CW), F32)] * SNB + [
            pltpu.SemaphoreType.DMA,
            pltpu.SemaphoreType.DMA,
            pltpu.SemaphoreType.DMA,
            pltpu.SemaphoreType.DMA,
            pltpu.VMEM_SHARED((N, CW), F32),
        ],
    )
    return k(m, dst2d, h)


def _pool_body(hp_hbm, idx_hbm, sums_hbm, cnts_hbm,
               idxb, hbuf, zbuf, obuf,
               semA, semB, semC, semD, accS, accC):
    c = lax.axis_index("c")
    s = lax.axis_index("s")

    @pl.loop(0, EB)
    def _(i):
        @pl.loop(0, D // 16)
        def _(j):
            zbuf[pl.ds(i, 1), pl.ds(j * 16, 16)] = jnp.zeros((1, 16), F32)

    @pl.loop(0, EB)
    def _(i):
        obuf[pl.ds(i, 1), pl.ds(0, 16)] = jnp.ones((1, 16), F32)

    @pl.loop(0, POOL_PS // EB)
    def _(i):
        pltpu.sync_copy(zbuf, accS.at[pl.ds(s * POOL_PS + i * EB, EB)])
        pltpu.sync_copy(zbuf.at[pl.ds(0, EB), pl.ds(0, 16)],
                        accC.at[pl.ds(s * POOL_PS + i * EB, EB)])
    plsc.subcore_barrier()
    hbuf2 = zbuf

    base = (c * NS + s) * PBPS
    pltpu.sync_copy(idx_hbm.at[pl.ds(base, PBPS)], idxb)

    def _href(i):
        return hp_hbm.at[pl.ds((base + i) * EB, EB)]

    bufs = (hbuf, hbuf2)
    rsems = (semA, semB)
    asems = (semC, semD)

    for b in range(2):
        pltpu.async_copy(_href(b), bufs[b], rsems[b])

    @pl.loop(0, PBPS // 2 - 1)
    def _(r):
        k = r * 2
        for b in range(2):
            pltpu.make_async_copy(_href(k + b), bufs[b], rsems[b]).wait()
            pltpu.async_copy(bufs[b], accS.at[idxb.at[k + b]], asems[b],
                             add=True)
            pltpu.async_copy(obuf, accC.at[idxb.at[k + b]], asems[b],
                             add=True)
        for b in range(2):
            pltpu.make_async_copy(bufs[b], accS.at[idxb.at[k + b]],
                                  asems[b]).wait()
            pltpu.make_async_copy(obuf, accC.at[idxb.at[k + b]],
                                  asems[b]).wait()
            pltpu.async_copy(_href(k + 2 + b), bufs[b], rsems[b])

    k = (PBPS // 2 - 1) * 2
    for b in range(2):
        pltpu.make_async_copy(_href(k + b), bufs[b], rsems[b]).wait()
        pltpu.async_copy(bufs[b], accS.at[idxb.at[k + b]], asems[b], add=True)
        pltpu.async_copy(obuf, accC.at[idxb.at[k + b]], asems[b], add=True)
    for b in range(2):
        pltpu.make_async_copy(bufs[b], accS.at[idxb.at[k + b]], asems[b]).wait()
        pltpu.make_async_copy(obuf, accC.at[idxb.at[k + b]], asems[b]).wait()

    plsc.subcore_barrier()
    pltpu.sync_copy(
        accS.at[pl.ds(s * POOL_PS, POOL_PS)],
        sums_hbm.at[c].at[pl.ds(s * POOL_PS, POOL_PS)],
    )
    pltpu.sync_copy(
        accC.at[pl.ds(s * POOL_PS, POOL_PS)],
        cnts_hbm.at[c].at[pl.ds(s * POOL_PS, POOL_PS)],
    )


def _sc_pool(hp, idx2d):
    k = pl.kernel(
        _pool_body,
        out_type=(
            jax.ShapeDtypeStruct((NC, NPOOL, D), F32),
            jax.ShapeDtypeStruct((NC, NPOOL, 16), F32),
        ),
        mesh=_sc_mesh(),
        compiler_params=_SC_PARAMS,
        scratch_types=[
            pltpu.VMEM((PBPS, EB), jnp.int32),
            pltpu.VMEM((EB, D), F32),
            pltpu.VMEM((EB, D), F32),
            pltpu.VMEM((EB, 16), F32),
            pltpu.SemaphoreType.DMA,
            pltpu.SemaphoreType.DMA,
            pltpu.SemaphoreType.DMA,
            pltpu.SemaphoreType.DMA,
            pltpu.VMEM_SHARED((NPOOL, D), F32),
            pltpu.VMEM_SHARED((NPOOL, 16), F32),
        ],
    )
    return k(hp, idx2d)


def _mm_bias_body(x_ref, w_ref, b_ref, o_ref):
    o_ref[...] = (
        jnp.dot(x_ref[...], w_ref[...], preferred_element_type=F32) + b_ref[...]
    )


def _tc_mm_bias(x, w, b, blk):
    rows = x.shape[0]
    return pl.pallas_call(
        _mm_bias_body,
        grid=(rows // blk,),
        in_specs=[
            pl.BlockSpec((blk, x.shape[1]), lambda i: (i, 0)),
            pl.BlockSpec(w.shape, lambda i: (0, 0)),
            pl.BlockSpec((1, w.shape[1]), lambda i: (0, 0)),
        ],
        out_specs=pl.BlockSpec((blk, w.shape[1]), lambda i: (i, 0)),
        out_shape=jax.ShapeDtypeStruct((rows, w.shape[1]), F32),
    )(x, w, b)


def _edge_proj_body(a_ref, w_ref, b_ref, o_ref):
    o_ref[...] = jnp.maximum(
        jnp.dot(a_ref[...], w_ref[...], preferred_element_type=F32) + b_ref[...],
        0.0,
    )


def _tc_edge_proj(attr_pad, w_all, b_all):
    blk = 512
    return pl.pallas_call(
        _edge_proj_body,
        grid=(E_PAD // blk,),
        in_specs=[
            pl.BlockSpec((blk, attr_pad.shape[1]), lambda i: (i, 0)),
            pl.BlockSpec(w_all.shape, lambda i: (0, 0)),
            pl.BlockSpec((1, w_all.shape[1]), lambda i: (0, 0)),
        ],
        out_specs=pl.BlockSpec((blk, w_all.shape[1]), lambda i: (i, 0)),
        out_shape=jax.ShapeDtypeStruct((E_PAD, w_all.shape[1]), F32),
    )(attr_pad, w_all, b_all)


def _msg_body(nreal_blocks, g_ref, e_ref, o_ref):
    v = jnp.maximum(g_ref[...] + e_ref[...], 0.0)
    o_ref[...] = jnp.where(pl.program_id(0) < nreal_blocks, v, 0.0)


def _tc_messages(g, e_all, layer):
    blk = 512
    nreal = E // blk
    return pl.pallas_call(
        functools.partial(_msg_body, nreal),
        grid=(E_PAD // blk,),
        in_specs=[
            pl.BlockSpec((blk, D), lambda i: (i, 0)),
            pl.BlockSpec((blk, D), lambda i, L=layer: (i, L)),
        ],
        out_specs=pl.BlockSpec((blk, D), lambda i: (i, 0)),
        out_shape=jax.ShapeDtypeStruct((E_PAD, D), F32),
    )(g, e_all)


def _mlp_body(z_ref, w1_ref, b1_ref, w2_ref, b2_ref, o_ref):
    t = jnp.maximum(
        jnp.dot(z_ref[...], w1_ref[...], preferred_element_type=F32)
        + b1_ref[...],
        0.0,
    )
    o_ref[...] = jnp.maximum(
        jnp.dot(t, w2_ref[...], preferred_element_type=F32) + b2_ref[...],
        0.0,
    )


def _tc_mlp(z, w1, b1, w2, b2, blk):
    rows = z.shape[0]
    return pl.pallas_call(
        _mlp_body,
        grid=(rows // blk,),
        in_specs=[
            pl.BlockSpec((blk, D), lambda i: (i, 0)),
            pl.BlockSpec((D, D), lambda i: (0, 0)),
            pl.BlockSpec((1, D), lambda i: (0, 0)),
            pl.BlockSpec((D, D), lambda i: (0, 0)),
            pl.BlockSpec((1, D), lambda i: (0, 0)),
        ],
        out_specs=pl.BlockSpec((blk, D), lambda i: (i, 0)),
        out_shape=jax.ShapeDtypeStruct((rows, D), F32),
    )(z, w1, b1, w2, b2)


def _final_body(s_ref, c_ref, w_ref, b_ref, o_ref):
    ssum = s_ref[0] + s_ref[1]
    cnt = c_ref[0, :, 0:1] + c_ref[1, :, 0:1]
    pooled = ssum / jnp.maximum(cnt, 1.0)
    o_ref[...] = (
        jnp.dot(pooled, w_ref[...], preferred_element_type=F32) + b_ref[...]
    )


def _tc_final(sums, cnts, w_out, b_out):
    blk = 400
    return pl.pallas_call(
        _final_body,
        grid=(NORIG // blk,),
        in_specs=[
            pl.BlockSpec((NC, blk, D), lambda i: (0, i, 0)),
            pl.BlockSpec((NC, blk, 16), lambda i: (0, i, 0)),
            pl.BlockSpec((D, D), lambda i: (0, 0)),
            pl.BlockSpec((1, D), lambda i: (0, 0)),
        ],
        out_specs=pl.BlockSpec((blk, D), lambda i: (i, 0)),
        out_shape=jax.ShapeDtypeStruct((NORIG, D), F32),
    )(sums, cnts, w_out, b_out)


def kernel(x, edge_index, edge_attr, node2orig, W_in, b_in, W_edge, b_edge,
           W_mlp, b_mlp, W_out, b_out):
    n_layers = W_edge.shape[0]

    pad_e = E_PAD - E
    src = jnp.concatenate(
        [edge_index[0], jnp.zeros((pad_e,), jnp.int32)]).reshape(NBLK_E, EB)
    dst = jnp.concatenate(
        [edge_index[1], jnp.zeros((pad_e,), jnp.int32)]).reshape(SBLK, SEB)
    attr_pad = jnp.concatenate(
        [edge_attr, jnp.zeros((pad_e, edge_attr.shape[1]), F32)])

    n2o = jnp.concatenate(
        [node2orig, jnp.full((N_PAD - N,), NORIG, jnp.int32)]).reshape(
            NBLK_P, EB)

    w_edge_all = W_edge.transpose(1, 0, 2).reshape(W_edge.shape[1],
                                                   n_layers * D)
    b_edge_all = b_edge.reshape(1, n_layers * D)

    h = _tc_mm_bias(x, W_in, b_in.reshape(1, D), 400)
    e_all = _tc_edge_proj(attr_pad, w_edge_all, b_edge_all)

    for l in range(n_layers):
        g = _sc_gather(h, src)
        m = _tc_messages(g, e_all, l)
        out_rows = N if l < n_layers - 1 else N_PAD
        z = _sc_scatter_z(m, dst, h, out_rows)
        blk = 400 if l < n_layers - 1 else 512
        h = _tc_mlp(z, W_mlp[l, 0], b_mlp[l, 0].reshape(1, D),
                    W_mlp[l, 1], b_mlp[l, 1].reshape(1, D), blk)

    sums, cnts = _sc_pool(h, n2o)
    return _tc_final(sums, cnts, W_out, b_out.reshape(1, D))
